# baseline (speedup 1.0000x reference)
import functools

import jax
import jax.numpy as jnp
from jax import lax
from jax.experimental import pallas as pl
from jax.experimental.pallas import tpu as pltpu
from jax.experimental.pallas import tpu_sc as plsc

_ECONF = (
    (0.0, 0.0, 0.0, 0.0),
    (1.0, 1.0, 0.0, 0.0),
    (2.0, 2.0, 0.0, 0.0),
    (3.0, 2.0, 1.0, 0.0),
    (4.0, 2.0, 2.0, 0.0),
    (5.0, 2.0, 2.0, 1.0),
    (6.0, 2.0, 2.0, 2.0),
    (7.0, 2.0, 2.0, 3.0),
    (8.0, 2.0, 2.0, 4.0),
    (9.0, 2.0, 2.0, 5.0),
)

_ROWS = 10
_D = 256
_N = 100000
_K = 128
_REP = 1024
_NW = 32
_NBUF = 3
_NIB = 6
_LANES = 16
_FULL_CHUNKS = _N // _K
_TAIL = _N - _FULL_CHUNKS * _K
_TAIL_BASE = _FULL_CHUNKS * _K
_TAIL_WID = _FULL_CHUNKS % _NW
_ITERS = -(-_FULL_CHUNKS // _NW)


def _table_body(econf_ref, emb_ref, wt_ref, out_ref):
    acc = emb_ref[...]
    for k in range(4):
        acc = acc + econf_ref[:, k : k + 1] * wt_ref[k : k + 1, :]
    out_ref[...] = jnp.broadcast_to(acc[None], (_REP, _ROWS, _D))


def _build_table(econf, emb, wt):
    return pl.pallas_call(
        _table_body,
        out_shape=jax.ShapeDtypeStruct((_REP, _ROWS, _D), jnp.float32),
    )(econf, emb, wt)


_mesh = plsc.VectorSubcoreMesh(core_axis_name="c", subcore_axis_name="s")


@functools.partial(
    pl.kernel,
    out_type=jax.ShapeDtypeStruct((_N, _D), jnp.float32),
    mesh=_mesh,
    scratch_types=[
        pltpu.VMEM((_NIB, _K), jnp.int32),
        pltpu.VMEM((_NBUF, _K, _D), jnp.float32),
        [pltpu.SemaphoreType.DMA] * _NIB,
        [pltpu.SemaphoreType.DMA] * _NBUF,
        [pltpu.SemaphoreType.DMA] * _NBUF,
    ],
)
def _gather_kernel(table_hbm, z_hbm, out_hbm, idx_v, rows_v, zsem, gsem, wsem):
    wid = lax.axis_index("s") * 2 + lax.axis_index("c")

    def chunk_of(t):
        return wid + _NW * t

    def zcopy(t):
        i = t % _NIB
        base = pl.multiple_of(chunk_of(t) * _K, _K)
        return pltpu.make_async_copy(
            z_hbm.at[pl.ds(base, _K)], idx_v.at[i], zsem[i]
        )

    def wcopy(t):
        b = t % _NBUF
        base = pl.multiple_of(chunk_of(t) * _K, _K)
        return pltpu.make_async_copy(
            rows_v.at[b], out_hbm.at[pl.ds(base, _K)], wsem[b]
        )

    def gcopy(t):
        return pltpu.make_async_copy(
            table_hbm.at[idx_v.at[t % _NIB]],
            rows_v.at[t % _NBUF],
            gsem[t % _NBUF],
        )

    def spread(i, n, t):
        rot = (t % 8) * _K
        for j in range(n // _LANES):
            sl = pl.ds(j * _LANES, _LANES)
            off = (lax.iota(jnp.int32, _LANES) + (j * _LANES) + rot) * _ROWS
            idx_v.at[i][sl] = idx_v.at[i][sl] + off

    def prefetch(t):
        if t >= _ITERS:
            return

        @pl.when(chunk_of(t) < _FULL_CHUNKS)
        def _():
            zcopy(t).start()

    def fire(t):
        @pl.when(chunk_of(t) < _FULL_CHUNKS)
        def _():
            if t >= _NBUF:
                wcopy(t - _NBUF).wait()
            zcopy(t).wait()
            spread(t % _NIB, _K, t)
            gcopy(t).start()

    def drain(t):
        @pl.when(chunk_of(t) < _FULL_CHUNKS)
        def _():
            gcopy(t).wait()
            wcopy(t).start()

    def finish(t):
        @pl.when(chunk_of(t) < _FULL_CHUNKS)
        def _():
            wcopy(t).wait()

    for t in range(_NBUF):
        prefetch(t)
    fire(0)
    for t in range(1, _ITERS):
        prefetch(t + _NBUF - 1)
        fire(t)
        drain(t - 1)
    drain(_ITERS - 1)
    for t in range(max(0, _ITERS - _NBUF), _ITERS):
        finish(t)

    @pl.when(wid == _TAIL_WID)
    def _():
        pltpu.sync_copy(
            z_hbm.at[pl.ds(_TAIL_BASE, _TAIL)], idx_v.at[0].at[pl.ds(0, _TAIL)]
        )
        spread(0, _TAIL, 0)
        pltpu.async_copy(
            table_hbm.at[idx_v.at[0].at[pl.ds(0, _TAIL)]],
            rows_v.at[0].at[pl.ds(0, _TAIL)],
            gsem[0],
        ).wait()
        pltpu.sync_copy(
            rows_v.at[0].at[pl.ds(0, _TAIL)],
            out_hbm.at[pl.ds(_TAIL_BASE, _TAIL)],
        )


def kernel(z, element_embedding, W):
    econf = jnp.asarray(_ECONF, dtype=jnp.float32)
    table = _build_table(econf, element_embedding, W.T)
    table = table.reshape(_REP * _ROWS, _D)
    return _gather_kernel(table, z.astype(jnp.int32))


# speedup vs baseline: 1.0878x; 1.0878x over previous
import functools

import jax
import jax.numpy as jnp
from jax import lax
from jax.experimental import pallas as pl
from jax.experimental.pallas import tpu as pltpu
from jax.experimental.pallas import tpu_sc as plsc

_ECONF = (
    (0.0, 0.0, 0.0, 0.0),
    (1.0, 1.0, 0.0, 0.0),
    (2.0, 2.0, 0.0, 0.0),
    (3.0, 2.0, 1.0, 0.0),
    (4.0, 2.0, 2.0, 0.0),
    (5.0, 2.0, 2.0, 1.0),
    (6.0, 2.0, 2.0, 2.0),
    (7.0, 2.0, 2.0, 3.0),
    (8.0, 2.0, 2.0, 4.0),
    (9.0, 2.0, 2.0, 5.0),
)

_ROWS = 10
_D = 256
_N = 100000
_K = 128
_REP = 512
_NW = 32
_NBUF = 3
_NIB = 6
_LANES = 16
_FULL_CHUNKS = _N // _K
_TAIL = _N - _FULL_CHUNKS * _K
_TAIL_BASE = _FULL_CHUNKS * _K
_TAIL_WID = _FULL_CHUNKS % _NW
_ITERS = -(-_FULL_CHUNKS // _NW)


def _table_body(econf_ref, emb_ref, wt_ref, out_ref):
    acc = emb_ref[...]
    for k in range(4):
        acc = acc + econf_ref[:, k : k + 1] * wt_ref[k : k + 1, :]
    out_ref[...] = jnp.broadcast_to(acc[None], (_REP, _ROWS, _D))


def _build_table(econf, emb, wt):
    return pl.pallas_call(
        _table_body,
        out_shape=jax.ShapeDtypeStruct((_REP, _ROWS, _D), jnp.float32),
    )(econf, emb, wt)


_mesh = plsc.VectorSubcoreMesh(core_axis_name="c", subcore_axis_name="s")


@functools.partial(
    pl.kernel,
    out_type=jax.ShapeDtypeStruct((_N, _D), jnp.float32),
    mesh=_mesh,
    scratch_types=[
        pltpu.VMEM((_NIB, _K), jnp.int32),
        pltpu.VMEM((_NBUF, _K, _D), jnp.float32),
        [pltpu.SemaphoreType.DMA] * _NIB,
        [pltpu.SemaphoreType.DMA] * _NBUF,
        [pltpu.SemaphoreType.DMA] * _NBUF,
    ],
)
def _gather_kernel(table_hbm, z_hbm, out_hbm, idx_v, rows_v, zsem, gsem, wsem):
    wid = lax.axis_index("s") * 2 + lax.axis_index("c")

    def chunk_of(t):
        return wid + _NW * t

    def zcopy(t):
        i = t % _NIB
        base = pl.multiple_of(chunk_of(t) * _K, _K)
        return pltpu.make_async_copy(
            z_hbm.at[pl.ds(base, _K)], idx_v.at[i], zsem[i]
        )

    def wcopy(t):
        b = t % _NBUF
        base = pl.multiple_of(chunk_of(t) * _K, _K)
        return pltpu.make_async_copy(
            rows_v.at[b], out_hbm.at[pl.ds(base, _K)], wsem[b]
        )

    def gcopy(t):
        return pltpu.make_async_copy(
            table_hbm.at[idx_v.at[t % _NIB]],
            rows_v.at[t % _NBUF],
            gsem[t % _NBUF],
        )

    def spread(i, n, t):
        rot = (t % 4) * _K
        for j in range(n // _LANES):
            sl = pl.ds(j * _LANES, _LANES)
            off = (lax.iota(jnp.int32, _LANES) + (j * _LANES) + rot) * _ROWS
            idx_v.at[i][sl] = idx_v.at[i][sl] + off

    def prefetch(t):
        if t >= _ITERS:
            return

        @pl.when(chunk_of(t) < _FULL_CHUNKS)
        def _():
            zcopy(t).start()

    def fire(t):
        @pl.when(chunk_of(t) < _FULL_CHUNKS)
        def _():
            if t >= _NBUF:
                wcopy(t - _NBUF).wait()
            zcopy(t).wait()
            spread(t % _NIB, _K, t)
            gcopy(t).start()

    def drain(t):
        @pl.when(chunk_of(t) < _FULL_CHUNKS)
        def _():
            gcopy(t).wait()
            wcopy(t).start()

    def finish(t):
        @pl.when(chunk_of(t) < _FULL_CHUNKS)
        def _():
            wcopy(t).wait()

    for t in range(_NBUF):
        prefetch(t)
    fire(0)
    for t in range(1, _ITERS):
        prefetch(t + _NBUF - 1)
        fire(t)
        drain(t - 1)
    drain(_ITERS - 1)
    for t in range(max(0, _ITERS - _NBUF), _ITERS):
        finish(t)

    @pl.when(wid == _TAIL_WID)
    def _():
        pltpu.sync_copy(
            z_hbm.at[pl.ds(_TAIL_BASE, _TAIL)], idx_v.at[0].at[pl.ds(0, _TAIL)]
        )
        spread(0, _TAIL, 0)
        pltpu.async_copy(
            table_hbm.at[idx_v.at[0].at[pl.ds(0, _TAIL)]],
            rows_v.at[0].at[pl.ds(0, _TAIL)],
            gsem[0],
        ).wait()
        pltpu.sync_copy(
            rows_v.at[0].at[pl.ds(0, _TAIL)],
            out_hbm.at[pl.ds(_TAIL_BASE, _TAIL)],
        )


def kernel(z, element_embedding, W):
    econf = jnp.asarray(_ECONF, dtype=jnp.float32)
    table = _build_table(econf, element_embedding, W.T)
    table = table.reshape(_REP * _ROWS, _D)
    return _gather_kernel(table, z.astype(jnp.int32))
